# baseline (speedup 1.0000x reference)
import jax
import jax.numpy as jnp
from jax import lax
from jax.experimental import pallas as pl
from jax.experimental.pallas import tpu as pltpu
from jax.experimental.pallas import tpu_sc as plsc

N_OBJ = 4096
D = 256
DH = 128
N_REL = 131072
NT = 16
CHUNK = N_REL // NT
L = 16
GROUPS = CHUNK // L
ROWS_T = N_OBJ // NT


def _tc1_body(obj_ref, wsw_ref, wsb_ref, wow_ref, wob_ref, cw_ref, cb_ref,
              s_ref, o_ref, cf_ref):
    x = obj_ref[...]
    s_ref[...] = jnp.dot(x, wsw_ref[...], preferred_element_type=jnp.float32) + wsb_ref[...]
    o_ref[...] = jnp.dot(x, wow_ref[...], preferred_element_type=jnp.float32) + wob_ref[...]
    cf_ref[...] = jnp.maximum(
        jnp.dot(x, cw_ref[...], preferred_element_type=jnp.float32) + cb_ref[...], 0.0)


def _tc2_body(obj_ref, cf_ref, rs_ref, o1_ref, o2_ref, t1w_ref, t1b_ref,
              lng_ref, lnb_ref, t2w_ref, t2b_ref, out_ref):
    r = 2047.5 + rs_ref[...]
    cf = cf_ref[...]
    g = cf / r[:, None]
    sumg = jnp.sum(g, axis=0)
    sumc = jnp.sum(cf, axis=0)
    out1 = 0.5 * sumg[None, :] - 0.5 * g + o1_ref[...]
    out2 = (0.5 * sumc[None, :] - 0.5 * cf + o2_ref[...]) / r[:, None]
    ctx = jnp.concatenate([out1, out2], axis=1)
    h = jnp.dot(ctx, t1w_ref[...], preferred_element_type=jnp.float32) + t1b_ref[...]
    mu = jnp.mean(h, axis=1, keepdims=True)
    var = jnp.mean((h - mu) ** 2, axis=1, keepdims=True)
    h = (h - mu) / jnp.sqrt(var + 1e-5) * lng_ref[...] + lnb_ref[...]
    h = jnp.maximum(h, 0.0)
    y = jnp.dot(h, t2w_ref[...], preferred_element_type=jnp.float32) + t2b_ref[...]
    out_ref[...] = jnp.maximum(obj_ref[...] + y, 0.0)


SUBR = 8
ZCAP = N_REL // SUBR


def _sc_body(rel0_h, rel1_h, rel2_h, s_h, o_h, phr_h, convf_h, wwb_h,
             rsum_h, out1_h, out2_h, own_h, zsum_h,
             c0, lids, blist, clist,
             wwb_v, meta_v, metal, ids_i, iv_b, jv_b,
             sbuf, obuf, pbuf, abuf, bbuf, cbuf, cob,
             cfa, cfb, st1, st2, rv_b, z0, z1,
             spm_rs, spm_o1, spm_o2, spm_meta, spm_z,
             sem0, sem1, sem2):
    w = lax.axis_index("s")
    iota = lax.iota(jnp.int32, L)
    row0 = w * ROWS_T

    def zz0(i, _):
        rr = i // (DH // L)
        cc = (i % (DH // L)) * L
        z0[rr, pl.ds(cc, L)] = jnp.zeros((L,), jnp.float32)
        return 0
    lax.fori_loop(0, L * (DH // L), zz0, 0)

    def zz1(i, _):
        z1[pl.ds(i * L, L)] = jnp.zeros((L,), jnp.float32)
        return 0
    lax.fori_loop(0, ROWS_T // L, zz1, 0)

    pltpu.sync_copy(z1, spm_rs.at[pl.ds(row0, ROWS_T)])
    for blk in range(ROWS_T // L):
        pltpu.sync_copy(z0, spm_o1.at[pl.ds(row0 + blk * L, L)])
        pltpu.sync_copy(z0, spm_o2.at[pl.ds(row0 + blk * L, L)])

    pltpu.sync_copy(wwb_h, wwb_v)

    base = w * CHUNK
    pltpu.sync_copy(rel0_h.at[pl.ds(base, CHUNK)], c0)

    UNR = 4

    def scan_body(gi, k):
        ms = [c0[pl.ds((gi * UNR + u) * L, L)] == 0 for u in range(UNR)]
        cnts = [plsc.all_reduce_population_count(mu)[0] for mu in ms]
        tot = cnts[0] + cnts[1] + cnts[2] + cnts[3]

        @pl.when(tot > 0)
        def _():
            kk = k
            for u in range(UNR):
                @pl.when(cnts[u] > 0)
                def _(mu=ms[u], kk=kk, uu=u):
                    pos = kk + plsc.cumsum(mu.astype(jnp.int32)) - 1
                    ids = base + (gi * UNR + uu) * L + iota
                    plsc.store_scatter(lids, [pos], ids, mask=mu)
                kk = kk + cnts[u]

        return k + tot

    k_w = lax.fori_loop(0, GROUPS // UNR, scan_body, jnp.int32(0))
    ngrp = (k_w + (L - 1)) // L

    def b1_body(gi, vmin):
        off = gi * L
        valid = off + iota < k_w
        idv = jnp.where(valid, lids[pl.ds(off, L)], 0)
        ids_i[...] = idv
        cp_p = pltpu.async_copy(phr_h.at[ids_i], pbuf, sem0)
        cp_i = pltpu.async_copy(rel1_h.at[ids_i], iv_b, sem1)
        cp_j = pltpu.async_copy(rel2_h.at[ids_i], jv_b, sem2)
        cp_p.wait()
        cp_i.wait()
        cp_j.wait()
        cp_s = pltpu.async_copy(s_h.at[iv_b], sbuf, sem1)
        cp_o = pltpu.async_copy(o_h.at[jv_b], obuf, sem2)
        cp_s.wait()
        cp_o.wait()
        iv = iv_b[...]
        jv = jv_b[...]

        def dot_body(rr, zv):
            acc = jnp.zeros((L,), jnp.float32)
            for dd in range(D // L):
                acc = acc + (sbuf[rr, pl.ds(dd * L, L)]
                             * obuf[rr, pl.ds(dd * L, L)]
                             * pbuf[rr, pl.ds(dd * L, L)]
                             * wwb_v[pl.ds(dd * L, L)])
            return jnp.where(iota == rr, jnp.sum(acc), zv)

        zv = lax.fori_loop(0, L, dot_body, jnp.zeros((L,), jnp.float32))
        wbv = wwb_v[pl.ds(D, L)]
        zv = zv + wbv[0]
        mn = jnp.where(valid, jnp.minimum(iv, jv), jnp.int32(N_OBJ))
        vmin = jnp.minimum(vmin, mn)
        lids[pl.ds(off, L)] = iv
        blist[pl.ds(off, L)] = jv
        clist[pl.ds(off, L)] = zv
        return vmin

    vmin = lax.fori_loop(0, ngrp, b1_body,
                         jnp.full((L,), N_OBJ, jnp.int32))
    h_w = jnp.min(vmin)

    meta_v[...] = jnp.where(iota == 0, h_w, jnp.where(iota == 1, k_w, 0))
    pltpu.sync_copy(meta_v, spm_meta.at[w])
    plsc.subcore_barrier()
    pltpu.sync_copy(spm_meta, metal)
    head = jnp.int32(N_OBJ)
    pref = jnp.int32(0)
    ktot = jnp.int32(0)
    for u in range(NT):
        mrow = metal[u, :]
        head = jnp.minimum(head, mrow[0])
        ku = mrow[1]
        ktot = ktot + ku
        pref = pref + jnp.where(jnp.int32(u) < w, ku, 0)

    def dda_body(gi, _):
        off = gi * L
        valid = off + iota < k_w
        iv = lids[pl.ds(off, L)]
        jv = blist[pl.ds(off, L)]
        key = jnp.where(valid, iv * N_OBJ + jv, 0)
        abuf[...] = key
        bbuf[...] = jnp.where(valid, pref + off + iota, jnp.int32(N_REL - 1))
        pltpu.sync_copy(bbuf, own_h.at[abuf])
        return 0

    lax.fori_loop(0, ngrp, dda_body, 0)
    plsc.subcore_barrier()

    zbase = w * (ZCAP // NT)
    nrounds = (ktot + (ZCAP - 1)) // ZCAP

    def round_body(rnd, _):
        for q in range(ZCAP // NT // ROWS_T):
            pltpu.sync_copy(z1, spm_z.at[pl.ds(zbase + q * ROWS_T, ROWS_T)])
        plsc.subcore_barrier()

        def ddb_body(gi, _):
            off = gi * L
            valid = off + iota < k_w
            iv = lids[pl.ds(off, L)]
            jv = blist[pl.ds(off, L)]
            zv = clist[pl.ds(off, L)]
            key = jnp.where(valid, iv * N_OBJ + jv, 0)
            abuf[...] = key
            cp = pltpu.async_copy(own_h.at[abuf], ids_i, sem0)
            cp.wait()
            ov = ids_i[...]
            inr = (ov // ZCAP) == rnd
            bbuf[...] = jnp.where(inr, ov - rnd * ZCAP, 0)
            cbuf[...] = jnp.where(valid & inr, zv, 0.0)
            pltpu.sync_copy(cbuf, spm_z.at[bbuf], add=True)
            return 0

        lax.fori_loop(0, ngrp, ddb_body, 0)
        plsc.subcore_barrier()
        pltpu.sync_copy(spm_z.at[pl.ds(zbase, ZCAP // NT)],
                        zsum_h.at[pl.ds(rnd * ZCAP + zbase, ZCAP // NT)])
        return 0

    lax.fori_loop(0, nrounds, round_body, 0)
    plsc.subcore_barrier()

    def ddw_body(gi, _):
        off = gi * L
        valid = off + iota < k_w
        iv = lids[pl.ds(off, L)]
        jv = blist[pl.ds(off, L)]
        zv = clist[pl.ds(off, L)]
        key = jnp.where(valid, iv * N_OBJ + jv, 0)
        abuf[...] = key
        cp = pltpu.async_copy(own_h.at[abuf], ids_i, sem0)
        cp.wait()
        ov = ids_i[...]
        gid = pref + off + iota
        winner = valid & (ov == gid)
        bbuf[...] = jnp.where(winner, gid, 0)
        cp2 = pltpu.async_copy(zsum_h.at[bbuf], cbuf, sem1)
        cp2.wait()
        zs = cbuf[...]
        clist[pl.ds(off, L)] = jnp.where(winner, zs, zv)
        blist[pl.ds(off, L)] = jnp.where(winner, jv, iv)
        return 0

    lax.fori_loop(0, ngrp, ddw_body, 0)

    def b2_body(gi, _):
        off = gi * L
        valid = off + iota < k_w
        iv = lids[pl.ds(off, L)]
        jv = blist[pl.ds(off, L)]
        zv = clist[pl.ds(off, L)]
        av = iv - head
        bv = jv - head
        cv = 1.0 / (1.0 + jnp.exp(-zv)) - 0.5
        cv = jnp.where(valid & (av != bv), cv, 0.0)
        av = jnp.where(valid, av, 0)
        bv = jnp.where(valid, bv, 0)
        lids[pl.ds(off, L)] = av
        blist[pl.ds(off, L)] = bv
        clist[pl.ds(off, L)] = cv
        abuf[...] = av
        cbuf[...] = cv
        pltpu.sync_copy(cbuf, spm_rs.at[abuf], add=True)
        return 0

    lax.fori_loop(0, ngrp, b2_body, 0)
    plsc.subcore_barrier()
    pltpu.sync_copy(spm_rs.at[pl.ds(row0, ROWS_T)],
                    rsum_h.at[pl.ds(row0, ROWS_T)])
    plsc.subcore_barrier()


    def c_body(gi, _):
        off = gi * L
        av = lids[pl.ds(off, L)]
        bv = blist[pl.ds(off, L)]
        cv = clist[pl.ds(off, L)]
        abuf[...] = av
        bbuf[...] = bv
        cpa = pltpu.async_copy(convf_h.at[abuf], cfa, sem0)
        cpb = pltpu.async_copy(convf_h.at[bbuf], cfb, sem1)
        cpr = pltpu.async_copy(rsum_h.at[bbuf], rv_b, sem2)
        cpa.wait()
        cpb.wait()
        cpr.wait()
        rv = rv_b[...]
        co1v = cv / (2047.5 + rv)
        cob[pl.ds(0, L)] = co1v

        def row_body(rr, _):
            cvw = clist[pl.ds(off + rr, L)]
            c_rr = cvw[0]
            cow = cob[pl.ds(rr, L)]
            co1 = cow[0]
            for dd in range(DH // L):
                st1[rr, pl.ds(dd * L, L)] = cfb[rr, pl.ds(dd * L, L)] * co1
                st2[rr, pl.ds(dd * L, L)] = cfa[rr, pl.ds(dd * L, L)] * c_rr
            return 0

        lax.fori_loop(0, L, row_body, 0)
        cp1 = pltpu.async_copy(st1, spm_o1.at[abuf], sem0, add=True)
        cp2 = pltpu.async_copy(st2, spm_o2.at[bbuf], sem1, add=True)
        cp1.wait()
        cp2.wait()
        return 0

    lax.fori_loop(0, ngrp, c_body, 0)
    plsc.subcore_barrier()

    pltpu.sync_copy(spm_o1.at[pl.ds(row0, ROWS_T)],
                    out1_h.at[pl.ds(row0, ROWS_T)])
    pltpu.sync_copy(spm_o2.at[pl.ds(row0, ROWS_T)],
                    out2_h.at[pl.ds(row0, ROWS_T)])


def _sc_call(rel0, rel1, rel2, s, o, phr, convf, wwb):
    mesh = plsc.VectorSubcoreMesh(core_axis_name="c", subcore_axis_name="s",
                                  num_cores=1)
    f = pl.kernel(
        _sc_body, mesh=mesh,
        compiler_params=pltpu.CompilerParams(needs_layout_passes=False),
        out_type=[jax.ShapeDtypeStruct((N_OBJ,), jnp.float32),
                  jax.ShapeDtypeStruct((N_OBJ, DH), jnp.float32),
                  jax.ShapeDtypeStruct((N_OBJ, DH), jnp.float32),
                  jax.ShapeDtypeStruct((N_OBJ * N_OBJ,), jnp.int32),
                  jax.ShapeDtypeStruct((N_REL,), jnp.float32)],
        scratch_types=[
            pltpu.VMEM((CHUNK,), jnp.int32),
            pltpu.VMEM((CHUNK,), jnp.int32),
            pltpu.VMEM((CHUNK,), jnp.int32),
            pltpu.VMEM((CHUNK + L,), jnp.float32),
            pltpu.VMEM((D + L,), jnp.float32),
            pltpu.VMEM((L,), jnp.int32),
            pltpu.VMEM((NT, L), jnp.int32),
            pltpu.VMEM((L,), jnp.int32),
            pltpu.VMEM((L,), jnp.int32),
            pltpu.VMEM((L,), jnp.int32),
            pltpu.VMEM((L, D), jnp.float32),
            pltpu.VMEM((L, D), jnp.float32),
            pltpu.VMEM((L, D), jnp.float32),
            pltpu.VMEM((L,), jnp.int32),
            pltpu.VMEM((L,), jnp.int32),
            pltpu.VMEM((L,), jnp.float32),
            pltpu.VMEM((2 * L,), jnp.float32),
            pltpu.VMEM((L, DH), jnp.float32),
            pltpu.VMEM((L, DH), jnp.float32),
            pltpu.VMEM((L, DH), jnp.float32),
            pltpu.VMEM((L, DH), jnp.float32),
            pltpu.VMEM((L,), jnp.float32),
            pltpu.VMEM((L, DH), jnp.float32),
            pltpu.VMEM((ROWS_T,), jnp.float32),
            pltpu.VMEM_SHARED((N_OBJ,), jnp.float32),
            pltpu.VMEM_SHARED((N_OBJ, DH), jnp.float32),
            pltpu.VMEM_SHARED((N_OBJ, DH), jnp.float32),
            pltpu.VMEM_SHARED((NT, L), jnp.int32),
            pltpu.VMEM_SHARED((ZCAP,), jnp.float32),
            pltpu.SemaphoreType.DMA,
            pltpu.SemaphoreType.DMA,
            pltpu.SemaphoreType.DMA,
        ])
    return f(rel0, rel1, rel2, s, o, phr, convf, wwb)


def kernel(obj_feats, phr_feats, im_inds, rel_inds, ws_w, ws_b, wo_w, wo_b,
           w_w, w_b, conv_w, conv_b, t1_w, t1_b, ln_g, ln_b, t2_w, t2_b):
    s, o, convf = pl.pallas_call(
        _tc1_body,
        out_shape=[jax.ShapeDtypeStruct((N_OBJ, D), jnp.float32),
                   jax.ShapeDtypeStruct((N_OBJ, D), jnp.float32),
                   jax.ShapeDtypeStruct((N_OBJ, DH), jnp.float32)],
    )(obj_feats, ws_w, ws_b, wo_w, wo_b, conv_w, conv_b)

    rel0 = rel_inds[:, 0]
    rel1 = rel_inds[:, 1]
    rel2 = rel_inds[:, 2]
    wwb = jnp.concatenate([w_w[:, 0], w_b,
                           jnp.zeros((L - 1,), jnp.float32)])

    rsum, out1c, out2c, _own, _zs = _sc_call(rel0, rel1, rel2, s, o,
                                             phr_feats, convf, wwb)

    out = pl.pallas_call(
        _tc2_body,
        out_shape=jax.ShapeDtypeStruct((N_OBJ, D), jnp.float32),
    )(obj_feats, convf, rsum, out1c, out2c, t1_w, t1_b, ln_g, ln_b,
      t2_w, t2_b)
    return out


# speedup vs baseline: 1.0020x; 1.0020x over previous
import jax
import jax.numpy as jnp
from jax import lax
from jax.experimental import pallas as pl
from jax.experimental.pallas import tpu as pltpu
from jax.experimental.pallas import tpu_sc as plsc

N_OBJ = 4096
D = 256
DH = 128
N_REL = 131072
NT = 16
CHUNK = N_REL // NT
L = 16
GROUPS = CHUNK // L
ROWS_T = N_OBJ // NT


def _tc1_body(obj_ref, wsw_ref, wsb_ref, wow_ref, wob_ref, cw_ref, cb_ref,
              s_ref, o_ref, cf_ref):
    x = obj_ref[...]
    s_ref[...] = jnp.dot(x, wsw_ref[...], preferred_element_type=jnp.float32) + wsb_ref[...]
    o_ref[...] = jnp.dot(x, wow_ref[...], preferred_element_type=jnp.float32) + wob_ref[...]
    cf_ref[...] = jnp.maximum(
        jnp.dot(x, cw_ref[...], preferred_element_type=jnp.float32) + cb_ref[...], 0.0)


def _tc2_body(obj_ref, cf_ref, rs_ref, o1_ref, o2_ref, t1w_ref, t1b_ref,
              lng_ref, lnb_ref, t2w_ref, t2b_ref, out_ref):
    r = 2047.5 + rs_ref[...]
    cf = cf_ref[...]
    g = cf / r[:, None]
    sumg = jnp.sum(g, axis=0)
    sumc = jnp.sum(cf, axis=0)
    out1 = 0.5 * sumg[None, :] - 0.5 * g + o1_ref[...]
    out2 = (0.5 * sumc[None, :] - 0.5 * cf + o2_ref[...]) / r[:, None]
    ctx = jnp.concatenate([out1, out2], axis=1)
    h = jnp.dot(ctx, t1w_ref[...], preferred_element_type=jnp.float32) + t1b_ref[...]
    mu = jnp.mean(h, axis=1, keepdims=True)
    var = jnp.mean((h - mu) ** 2, axis=1, keepdims=True)
    h = (h - mu) / jnp.sqrt(var + 1e-5) * lng_ref[...] + lnb_ref[...]
    h = jnp.maximum(h, 0.0)
    y = jnp.dot(h, t2w_ref[...], preferred_element_type=jnp.float32) + t2b_ref[...]
    out_ref[...] = jnp.maximum(obj_ref[...] + y, 0.0)


SUBR = 8
ZCAP = N_REL // SUBR


def _sc_body(rel0_h, rel1_h, rel2_h, s_h, o_h, phr_h, convf_h, wwb_h,
             rsum_h, out1_h, out2_h, own_h, zsum_h,
             c0, lids, blist, clist,
             wwb_v, meta_v, metal, ids_i, iv_b, jv_b,
             sbuf, obuf, pbuf, abuf, bbuf, cbuf, cob,
             cfa, cfb, st1, st2, rv_b, z0, z1,
             spm_rs, spm_o1, spm_o2, spm_meta, spm_z,
             sem0, sem1, sem2):
    w = lax.axis_index("s")
    iota = lax.iota(jnp.int32, L)
    row0 = w * ROWS_T

    def zz0(i, _):
        rr = i // (DH // L)
        cc = (i % (DH // L)) * L
        z0[rr, pl.ds(cc, L)] = jnp.zeros((L,), jnp.float32)
        return 0
    lax.fori_loop(0, L * (DH // L), zz0, 0)

    def zz1(i, _):
        z1[pl.ds(i * L, L)] = jnp.zeros((L,), jnp.float32)
        return 0
    lax.fori_loop(0, ROWS_T // L, zz1, 0)

    pltpu.sync_copy(z1, spm_rs.at[pl.ds(row0, ROWS_T)])
    for blk in range(ROWS_T // L):
        pltpu.sync_copy(z0, spm_o1.at[pl.ds(row0 + blk * L, L)])
        pltpu.sync_copy(z0, spm_o2.at[pl.ds(row0 + blk * L, L)])

    pltpu.sync_copy(wwb_h, wwb_v)

    base = w * CHUNK
    pltpu.sync_copy(rel0_h.at[pl.ds(base, CHUNK)], c0)

    UNR = 4

    def scan_body(gi, k):
        ms = [c0[pl.ds((gi * UNR + u) * L, L)] == 0 for u in range(UNR)]
        cnts = [plsc.all_reduce_population_count(mu)[0] for mu in ms]
        tot = cnts[0] + cnts[1] + cnts[2] + cnts[3]

        @pl.when(tot > 0)
        def _():
            kk = k
            for u in range(UNR):
                @pl.when(cnts[u] > 0)
                def _(mu=ms[u], kk=kk, uu=u):
                    pos = kk + plsc.cumsum(mu.astype(jnp.int32)) - 1
                    ids = base + (gi * UNR + uu) * L + iota
                    plsc.store_scatter(lids, [pos], ids, mask=mu)
                kk = kk + cnts[u]

        return k + tot

    k_w = lax.fori_loop(0, GROUPS // UNR, scan_body, jnp.int32(0))
    ngrp = (k_w + (L - 1)) // L

    def b1_body(gi, vmin):
        off = gi * L
        valid = off + iota < k_w
        idv = jnp.where(valid, lids[pl.ds(off, L)], 0)
        ids_i[...] = idv
        cp_p = pltpu.async_copy(phr_h.at[ids_i], pbuf, sem0)
        cp_i = pltpu.async_copy(rel1_h.at[ids_i], iv_b, sem1)
        cp_j = pltpu.async_copy(rel2_h.at[ids_i], jv_b, sem2)
        cp_p.wait()
        cp_i.wait()
        cp_j.wait()
        cp_s = pltpu.async_copy(s_h.at[iv_b], sbuf, sem1)
        cp_o = pltpu.async_copy(o_h.at[jv_b], obuf, sem2)
        cp_s.wait()
        cp_o.wait()
        iv = iv_b[...]
        jv = jv_b[...]

        def dot_body(rr, zv):
            acc = jnp.zeros((L,), jnp.float32)
            for dd in range(D // L):
                acc = acc + (sbuf[rr, pl.ds(dd * L, L)]
                             * obuf[rr, pl.ds(dd * L, L)]
                             * pbuf[rr, pl.ds(dd * L, L)]
                             * wwb_v[pl.ds(dd * L, L)])
            return jnp.where(iota == rr, jnp.sum(acc), zv)

        zv = lax.fori_loop(0, L, dot_body, jnp.zeros((L,), jnp.float32))
        wbv = wwb_v[pl.ds(D, L)]
        zv = zv + wbv[0]
        mn = jnp.where(valid, jnp.minimum(iv, jv), jnp.int32(N_OBJ))
        vmin = jnp.minimum(vmin, mn)
        lids[pl.ds(off, L)] = iv
        blist[pl.ds(off, L)] = jv
        clist[pl.ds(off, L)] = zv
        return vmin

    vmin = lax.fori_loop(0, ngrp, b1_body,
                         jnp.full((L,), N_OBJ, jnp.int32))
    h_w = jnp.min(vmin)

    meta_v[...] = jnp.where(iota == 0, h_w, jnp.where(iota == 1, k_w, 0))
    pltpu.sync_copy(meta_v, spm_meta.at[w])
    plsc.subcore_barrier()
    pltpu.sync_copy(spm_meta, metal)
    head = jnp.int32(N_OBJ)
    pref = jnp.int32(0)
    ktot = jnp.int32(0)
    for u in range(NT):
        mrow = metal[u, :]
        head = jnp.minimum(head, mrow[0])
        ku = mrow[1]
        ktot = ktot + ku
        pref = pref + jnp.where(jnp.int32(u) < w, ku, 0)

    def dda_body(gi, _):
        off = gi * L
        valid = off + iota < k_w
        iv = lids[pl.ds(off, L)]
        jv = blist[pl.ds(off, L)]
        key = jnp.where(valid, iv * N_OBJ + jv, 0)
        abuf[...] = key
        bbuf[...] = jnp.where(valid, pref + off + iota, jnp.int32(N_REL - 1))
        pltpu.sync_copy(bbuf, own_h.at[abuf])
        return 0

    lax.fori_loop(0, ngrp, dda_body, 0)
    plsc.subcore_barrier()

    zbase = w * (ZCAP // NT)
    nrounds = (ktot + (ZCAP - 1)) // ZCAP

    def round_body(rnd, _):
        for q in range(ZCAP // NT // ROWS_T):
            pltpu.sync_copy(z1, spm_z.at[pl.ds(zbase + q * ROWS_T, ROWS_T)])
        plsc.subcore_barrier()

        def ddb_body(gi, _):
            off = gi * L
            valid = off + iota < k_w
            iv = lids[pl.ds(off, L)]
            jv = blist[pl.ds(off, L)]
            zv = clist[pl.ds(off, L)]
            key = jnp.where(valid, iv * N_OBJ + jv, 0)
            abuf[...] = key
            cp = pltpu.async_copy(own_h.at[abuf], ids_i, sem0)
            cp.wait()
            ov = ids_i[...]
            inr = (ov // ZCAP) == rnd
            bbuf[...] = jnp.where(inr, ov - rnd * ZCAP, 0)
            cbuf[...] = jnp.where(valid & inr, zv, 0.0)
            pltpu.sync_copy(cbuf, spm_z.at[bbuf], add=True)
            return 0

        lax.fori_loop(0, ngrp, ddb_body, 0)
        plsc.subcore_barrier()
        pltpu.sync_copy(spm_z.at[pl.ds(zbase, ZCAP // NT)],
                        zsum_h.at[pl.ds(rnd * ZCAP + zbase, ZCAP // NT)])
        return 0

    lax.fori_loop(0, nrounds, round_body, 0)
    plsc.subcore_barrier()

    def b2_body(gi, _):
        off = gi * L
        valid = off + iota < k_w
        iv = lids[pl.ds(off, L)]
        jv = blist[pl.ds(off, L)]
        zv = clist[pl.ds(off, L)]
        key = jnp.where(valid, iv * N_OBJ + jv, 0)
        abuf[...] = key
        cp = pltpu.async_copy(own_h.at[abuf], ids_i, sem0)
        cp.wait()
        ov = ids_i[...]
        gid = pref + off + iota
        winner = valid & (ov == gid)
        bbuf[...] = jnp.where(winner, gid, 0)
        cp2 = pltpu.async_copy(zsum_h.at[bbuf], rv_b, sem1)
        cp2.wait()
        zv = jnp.where(winner, rv_b[...], zv)
        jv = jnp.where(winner, jv, iv)
        av = iv - head
        bv = jv - head
        cv = 1.0 / (1.0 + jnp.exp(-zv)) - 0.5
        cv = jnp.where(valid & (av != bv), cv, 0.0)
        av = jnp.where(valid, av, 0)
        bv = jnp.where(valid, bv, 0)
        lids[pl.ds(off, L)] = av
        blist[pl.ds(off, L)] = bv
        clist[pl.ds(off, L)] = cv
        abuf[...] = av
        cbuf[...] = cv
        pltpu.sync_copy(cbuf, spm_rs.at[abuf], add=True)
        return 0

    lax.fori_loop(0, ngrp, b2_body, 0)
    plsc.subcore_barrier()
    pltpu.sync_copy(spm_rs.at[pl.ds(row0, ROWS_T)],
                    rsum_h.at[pl.ds(row0, ROWS_T)])
    plsc.subcore_barrier()


    def c_body(gi, _):
        off = gi * L
        av = lids[pl.ds(off, L)]
        bv = blist[pl.ds(off, L)]
        cv = clist[pl.ds(off, L)]
        abuf[...] = av
        bbuf[...] = bv
        cpa = pltpu.async_copy(convf_h.at[abuf], cfa, sem0)
        cpb = pltpu.async_copy(convf_h.at[bbuf], cfb, sem1)
        cpr = pltpu.async_copy(rsum_h.at[bbuf], rv_b, sem2)
        cpa.wait()
        cpb.wait()
        cpr.wait()
        rv = rv_b[...]
        co1v = cv / (2047.5 + rv)
        cob[pl.ds(0, L)] = co1v

        def row_body(rr, _):
            cvw = clist[pl.ds(off + rr, L)]
            c_rr = cvw[0]
            cow = cob[pl.ds(rr, L)]
            co1 = cow[0]
            for dd in range(DH // L):
                st1[rr, pl.ds(dd * L, L)] = cfb[rr, pl.ds(dd * L, L)] * co1
                st2[rr, pl.ds(dd * L, L)] = cfa[rr, pl.ds(dd * L, L)] * c_rr
            return 0

        lax.fori_loop(0, L, row_body, 0)
        cp1 = pltpu.async_copy(st1, spm_o1.at[abuf], sem0, add=True)
        cp2 = pltpu.async_copy(st2, spm_o2.at[bbuf], sem1, add=True)
        cp1.wait()
        cp2.wait()
        return 0

    lax.fori_loop(0, ngrp, c_body, 0)
    plsc.subcore_barrier()

    pltpu.sync_copy(spm_o1.at[pl.ds(row0, ROWS_T)],
                    out1_h.at[pl.ds(row0, ROWS_T)])
    pltpu.sync_copy(spm_o2.at[pl.ds(row0, ROWS_T)],
                    out2_h.at[pl.ds(row0, ROWS_T)])


def _sc_call(rel0, rel1, rel2, s, o, phr, convf, wwb):
    mesh = plsc.VectorSubcoreMesh(core_axis_name="c", subcore_axis_name="s",
                                  num_cores=1)
    f = pl.kernel(
        _sc_body, mesh=mesh,
        compiler_params=pltpu.CompilerParams(needs_layout_passes=False),
        out_type=[jax.ShapeDtypeStruct((N_OBJ,), jnp.float32),
                  jax.ShapeDtypeStruct((N_OBJ, DH), jnp.float32),
                  jax.ShapeDtypeStruct((N_OBJ, DH), jnp.float32),
                  jax.ShapeDtypeStruct((N_OBJ * N_OBJ,), jnp.int32),
                  jax.ShapeDtypeStruct((N_REL,), jnp.float32)],
        scratch_types=[
            pltpu.VMEM((CHUNK,), jnp.int32),
            pltpu.VMEM((CHUNK,), jnp.int32),
            pltpu.VMEM((CHUNK,), jnp.int32),
            pltpu.VMEM((CHUNK + L,), jnp.float32),
            pltpu.VMEM((D + L,), jnp.float32),
            pltpu.VMEM((L,), jnp.int32),
            pltpu.VMEM((NT, L), jnp.int32),
            pltpu.VMEM((L,), jnp.int32),
            pltpu.VMEM((L,), jnp.int32),
            pltpu.VMEM((L,), jnp.int32),
            pltpu.VMEM((L, D), jnp.float32),
            pltpu.VMEM((L, D), jnp.float32),
            pltpu.VMEM((L, D), jnp.float32),
            pltpu.VMEM((L,), jnp.int32),
            pltpu.VMEM((L,), jnp.int32),
            pltpu.VMEM((L,), jnp.float32),
            pltpu.VMEM((2 * L,), jnp.float32),
            pltpu.VMEM((L, DH), jnp.float32),
            pltpu.VMEM((L, DH), jnp.float32),
            pltpu.VMEM((L, DH), jnp.float32),
            pltpu.VMEM((L, DH), jnp.float32),
            pltpu.VMEM((L,), jnp.float32),
            pltpu.VMEM((L, DH), jnp.float32),
            pltpu.VMEM((ROWS_T,), jnp.float32),
            pltpu.VMEM_SHARED((N_OBJ,), jnp.float32),
            pltpu.VMEM_SHARED((N_OBJ, DH), jnp.float32),
            pltpu.VMEM_SHARED((N_OBJ, DH), jnp.float32),
            pltpu.VMEM_SHARED((NT, L), jnp.int32),
            pltpu.VMEM_SHARED((ZCAP,), jnp.float32),
            pltpu.SemaphoreType.DMA,
            pltpu.SemaphoreType.DMA,
            pltpu.SemaphoreType.DMA,
        ])
    return f(rel0, rel1, rel2, s, o, phr, convf, wwb)


def kernel(obj_feats, phr_feats, im_inds, rel_inds, ws_w, ws_b, wo_w, wo_b,
           w_w, w_b, conv_w, conv_b, t1_w, t1_b, ln_g, ln_b, t2_w, t2_b):
    s, o, convf = pl.pallas_call(
        _tc1_body,
        out_shape=[jax.ShapeDtypeStruct((N_OBJ, D), jnp.float32),
                   jax.ShapeDtypeStruct((N_OBJ, D), jnp.float32),
                   jax.ShapeDtypeStruct((N_OBJ, DH), jnp.float32)],
    )(obj_feats, ws_w, ws_b, wo_w, wo_b, conv_w, conv_b)

    rel0 = rel_inds[:, 0]
    rel1 = rel_inds[:, 1]
    rel2 = rel_inds[:, 2]
    wwb = jnp.concatenate([w_w[:, 0], w_b,
                           jnp.zeros((L - 1,), jnp.float32)])

    rsum, out1c, out2c, _own, _zs = _sc_call(rel0, rel1, rel2, s, o,
                                             phr_feats, convf, wwb)

    out = pl.pallas_call(
        _tc2_body,
        out_shape=jax.ShapeDtypeStruct((N_OBJ, D), jnp.float32),
    )(obj_feats, convf, rsum, out1c, out2c, t1_w, t1_b, ln_g, ln_b,
      t2_w, t2_b)
    return out


# speedup vs baseline: 1.0200x; 1.0179x over previous
import jax
import jax.numpy as jnp
from jax import lax
from jax.experimental import pallas as pl
from jax.experimental.pallas import tpu as pltpu
from jax.experimental.pallas import tpu_sc as plsc

N_OBJ = 4096
D = 256
DH = 128
N_REL = 131072
NT = 16
CHUNK = N_REL // NT
L = 16
GROUPS = CHUNK // L
ROWS_T = N_OBJ // NT


def _tc1_body(obj_ref, wsw_ref, wsb_ref, wow_ref, wob_ref, cw_ref, cb_ref,
              s_ref, o_ref, cf_ref):
    x = obj_ref[...]
    s_ref[...] = jnp.dot(x, wsw_ref[...], preferred_element_type=jnp.float32) + wsb_ref[...]
    o_ref[...] = jnp.dot(x, wow_ref[...], preferred_element_type=jnp.float32) + wob_ref[...]
    cf_ref[...] = jnp.maximum(
        jnp.dot(x, cw_ref[...], preferred_element_type=jnp.float32) + cb_ref[...], 0.0)


def _tc2_body(obj_ref, cf_ref, rs_ref, o1_ref, o2_ref, t1w_ref, t1b_ref,
              lng_ref, lnb_ref, t2w_ref, t2b_ref, out_ref):
    r = 2047.5 + rs_ref[...]
    cf = cf_ref[...]
    g = cf / r[:, None]
    sumg = jnp.sum(g, axis=0)
    sumc = jnp.sum(cf, axis=0)
    out1 = 0.5 * sumg[None, :] - 0.5 * g + o1_ref[...]
    out2 = (0.5 * sumc[None, :] - 0.5 * cf + o2_ref[...]) / r[:, None]
    ctx = jnp.concatenate([out1, out2], axis=1)
    h = jnp.dot(ctx, t1w_ref[...], preferred_element_type=jnp.float32) + t1b_ref[...]
    mu = jnp.mean(h, axis=1, keepdims=True)
    var = jnp.mean((h - mu) ** 2, axis=1, keepdims=True)
    h = (h - mu) / jnp.sqrt(var + 1e-5) * lng_ref[...] + lnb_ref[...]
    h = jnp.maximum(h, 0.0)
    y = jnp.dot(h, t2w_ref[...], preferred_element_type=jnp.float32) + t2b_ref[...]
    out_ref[...] = jnp.maximum(obj_ref[...] + y, 0.0)


SUBR = 8
ZCAP = N_REL // SUBR


def _sc_body(rel0_h, rel1_h, rel2_h, s_h, o_h, phr_h, convf_h, wwb_h,
             rsum_h, out1_h, out2_h, own_h, zsum_h,
             c0, lids, blist, clist,
             wwb_v, meta_v, metal, ids_i, iv_b, jv_b,
             sbuf, obuf, pbuf, abuf, bbuf, cbuf, cob,
             cfa, cfb, st1, st2, rv_b, z0, z1,
             spm_rs, spm_o1, spm_o2, spm_meta, spm_z,
             sem0, sem1, sem2):
    w = lax.axis_index("s")
    iota = lax.iota(jnp.int32, L)
    row0 = w * ROWS_T

    def zz0(i, _):
        rr = i // (DH // L)
        cc = (i % (DH // L)) * L
        z0[rr, pl.ds(cc, L)] = jnp.zeros((L,), jnp.float32)
        return 0
    lax.fori_loop(0, L * (DH // L), zz0, 0)

    def zz1(i, _):
        z1[pl.ds(i * L, L)] = jnp.zeros((L,), jnp.float32)
        return 0
    lax.fori_loop(0, ROWS_T // L, zz1, 0)

    cps = [pltpu.async_copy(z1, spm_rs.at[pl.ds(row0, ROWS_T)], sem0)]
    for blk in range(ROWS_T // L):
        cps.append(pltpu.async_copy(z0, spm_o1.at[pl.ds(row0 + blk * L, L)],
                                    sem0))
        cps.append(pltpu.async_copy(z0, spm_o2.at[pl.ds(row0 + blk * L, L)],
                                    sem0))
    for cp in cps:
        cp.wait()

    pltpu.sync_copy(wwb_h, wwb_v)

    base = w * CHUNK
    pltpu.sync_copy(rel0_h.at[pl.ds(base, CHUNK)], c0)

    UNR = 4

    def scan_body(gi, k):
        ms = [c0[pl.ds((gi * UNR + u) * L, L)] == 0 for u in range(UNR)]
        cnts = [plsc.all_reduce_population_count(mu)[0] for mu in ms]
        tot = cnts[0] + cnts[1] + cnts[2] + cnts[3]

        @pl.when(tot > 0)
        def _():
            kk = k
            for u in range(UNR):
                @pl.when(cnts[u] > 0)
                def _(mu=ms[u], kk=kk, uu=u):
                    pos = kk + plsc.cumsum(mu.astype(jnp.int32)) - 1
                    ids = base + (gi * UNR + uu) * L + iota
                    plsc.store_scatter(lids, [pos], ids, mask=mu)
                kk = kk + cnts[u]

        return k + tot

    k_w = lax.fori_loop(0, GROUPS // UNR, scan_body, jnp.int32(0))
    ngrp = (k_w + (L - 1)) // L

    def b1_body(gi, vmin):
        off = gi * L
        valid = off + iota < k_w
        idv = jnp.where(valid, lids[pl.ds(off, L)], 0)
        ids_i[...] = idv
        cp_p = pltpu.async_copy(phr_h.at[ids_i], pbuf, sem0)
        cp_i = pltpu.async_copy(rel1_h.at[ids_i], iv_b, sem1)
        cp_j = pltpu.async_copy(rel2_h.at[ids_i], jv_b, sem2)
        cp_p.wait()
        cp_i.wait()
        cp_j.wait()
        cp_s = pltpu.async_copy(s_h.at[iv_b], sbuf, sem1)
        cp_o = pltpu.async_copy(o_h.at[jv_b], obuf, sem2)
        cp_s.wait()
        cp_o.wait()
        iv = iv_b[...]
        jv = jv_b[...]

        def dot_body(rr, zv):
            acc = jnp.zeros((L,), jnp.float32)
            for dd in range(D // L):
                acc = acc + (sbuf[rr, pl.ds(dd * L, L)]
                             * obuf[rr, pl.ds(dd * L, L)]
                             * pbuf[rr, pl.ds(dd * L, L)]
                             * wwb_v[pl.ds(dd * L, L)])
            return jnp.where(iota == rr, jnp.sum(acc), zv)

        zv = lax.fori_loop(0, L, dot_body, jnp.zeros((L,), jnp.float32))
        wbv = wwb_v[pl.ds(D, L)]
        zv = zv + wbv[0]
        mn = jnp.where(valid, jnp.minimum(iv, jv), jnp.int32(N_OBJ))
        vmin = jnp.minimum(vmin, mn)
        lids[pl.ds(off, L)] = iv
        blist[pl.ds(off, L)] = jv
        clist[pl.ds(off, L)] = zv
        return vmin

    vmin = lax.fori_loop(0, ngrp, b1_body,
                         jnp.full((L,), N_OBJ, jnp.int32))
    h_w = jnp.min(vmin)

    meta_v[...] = jnp.where(iota == 0, h_w, jnp.where(iota == 1, k_w, 0))
    pltpu.sync_copy(meta_v, spm_meta.at[w])
    plsc.subcore_barrier()
    pltpu.sync_copy(spm_meta, metal)
    head = jnp.int32(N_OBJ)
    pref = jnp.int32(0)
    ktot = jnp.int32(0)
    for u in range(NT):
        mrow = metal[u, :]
        head = jnp.minimum(head, mrow[0])
        ku = mrow[1]
        ktot = ktot + ku
        pref = pref + jnp.where(jnp.int32(u) < w, ku, 0)

    def dda_body(gi, _):
        off = gi * L
        valid = off + iota < k_w
        iv = lids[pl.ds(off, L)]
        jv = blist[pl.ds(off, L)]
        key = jnp.where(valid, iv * N_OBJ + jv, 0)
        abuf[...] = key
        bbuf[...] = jnp.where(valid, pref + off + iota, jnp.int32(N_REL - 1))
        pltpu.sync_copy(bbuf, own_h.at[abuf])
        return 0

    lax.fori_loop(0, ngrp, dda_body, 0)
    plsc.subcore_barrier()

    zbase = w * (ZCAP // NT)
    nrounds = (ktot + (ZCAP - 1)) // ZCAP

    def round_body(rnd, _):
        for q in range(ZCAP // NT // ROWS_T):
            pltpu.sync_copy(z1, spm_z.at[pl.ds(zbase + q * ROWS_T, ROWS_T)])
        plsc.subcore_barrier()

        def ddb_body(gi, _):
            off = gi * L
            valid = off + iota < k_w
            iv = lids[pl.ds(off, L)]
            jv = blist[pl.ds(off, L)]
            zv = clist[pl.ds(off, L)]
            key = jnp.where(valid, iv * N_OBJ + jv, 0)
            abuf[...] = key
            cp = pltpu.async_copy(own_h.at[abuf], ids_i, sem0)
            cp.wait()
            ov = ids_i[...]
            inr = (ov // ZCAP) == rnd
            bbuf[...] = jnp.where(inr, ov - rnd * ZCAP, 0)
            cbuf[...] = jnp.where(valid & inr, zv, 0.0)
            pltpu.sync_copy(cbuf, spm_z.at[bbuf], add=True)
            return 0

        lax.fori_loop(0, ngrp, ddb_body, 0)
        plsc.subcore_barrier()
        pltpu.sync_copy(spm_z.at[pl.ds(zbase, ZCAP // NT)],
                        zsum_h.at[pl.ds(rnd * ZCAP + zbase, ZCAP // NT)])
        return 0

    lax.fori_loop(0, nrounds, round_body, 0)
    plsc.subcore_barrier()

    def b2_body(gi, _):
        off = gi * L
        valid = off + iota < k_w
        iv = lids[pl.ds(off, L)]
        jv = blist[pl.ds(off, L)]
        zv = clist[pl.ds(off, L)]
        key = jnp.where(valid, iv * N_OBJ + jv, 0)
        abuf[...] = key
        cp = pltpu.async_copy(own_h.at[abuf], ids_i, sem0)
        cp.wait()
        ov = ids_i[...]
        gid = pref + off + iota
        winner = valid & (ov == gid)
        bbuf[...] = jnp.where(winner, gid, 0)
        cp2 = pltpu.async_copy(zsum_h.at[bbuf], rv_b, sem1)
        cp2.wait()
        zv = jnp.where(winner, rv_b[...], zv)
        jv = jnp.where(winner, jv, iv)
        av = iv - head
        bv = jv - head
        cv = 1.0 / (1.0 + jnp.exp(-zv)) - 0.5
        cv = jnp.where(valid & (av != bv), cv, 0.0)
        av = jnp.where(valid, av, 0)
        bv = jnp.where(valid, bv, 0)
        lids[pl.ds(off, L)] = av
        blist[pl.ds(off, L)] = bv
        clist[pl.ds(off, L)] = cv
        abuf[...] = av
        cbuf[...] = cv
        pltpu.sync_copy(cbuf, spm_rs.at[abuf], add=True)
        return 0

    lax.fori_loop(0, ngrp, b2_body, 0)
    plsc.subcore_barrier()
    pltpu.sync_copy(spm_rs.at[pl.ds(row0, ROWS_T)],
                    rsum_h.at[pl.ds(row0, ROWS_T)])
    plsc.subcore_barrier()


    def c_body(gi, _):
        off = gi * L
        av = lids[pl.ds(off, L)]
        bv = blist[pl.ds(off, L)]
        cv = clist[pl.ds(off, L)]
        abuf[...] = av
        bbuf[...] = bv
        cpa = pltpu.async_copy(convf_h.at[abuf], cfa, sem0)
        cpb = pltpu.async_copy(convf_h.at[bbuf], cfb, sem1)
        cpr = pltpu.async_copy(rsum_h.at[bbuf], rv_b, sem2)
        cpa.wait()
        cpb.wait()
        cpr.wait()
        rv = rv_b[...]
        co1v = cv / (2047.5 + rv)
        cob[pl.ds(0, L)] = co1v

        def row_body(rr, _):
            cvw = clist[pl.ds(off + rr, L)]
            c_rr = cvw[0]
            cow = cob[pl.ds(rr, L)]
            co1 = cow[0]
            for dd in range(DH // L):
                st1[rr, pl.ds(dd * L, L)] = cfb[rr, pl.ds(dd * L, L)] * co1
                st2[rr, pl.ds(dd * L, L)] = cfa[rr, pl.ds(dd * L, L)] * c_rr
            return 0

        lax.fori_loop(0, L, row_body, 0)
        cp1 = pltpu.async_copy(st1, spm_o1.at[abuf], sem0, add=True)
        cp2 = pltpu.async_copy(st2, spm_o2.at[bbuf], sem1, add=True)
        cp1.wait()
        cp2.wait()
        return 0

    lax.fori_loop(0, ngrp, c_body, 0)
    plsc.subcore_barrier()

    cpo1 = pltpu.async_copy(spm_o1.at[pl.ds(row0, ROWS_T)],
                            out1_h.at[pl.ds(row0, ROWS_T)], sem0)
    cpo2 = pltpu.async_copy(spm_o2.at[pl.ds(row0, ROWS_T)],
                            out2_h.at[pl.ds(row0, ROWS_T)], sem1)
    cpo1.wait()
    cpo2.wait()


def _sc_call(rel0, rel1, rel2, s, o, phr, convf, wwb):
    mesh = plsc.VectorSubcoreMesh(core_axis_name="c", subcore_axis_name="s",
                                  num_cores=1)
    f = pl.kernel(
        _sc_body, mesh=mesh,
        compiler_params=pltpu.CompilerParams(needs_layout_passes=False),
        out_type=[jax.ShapeDtypeStruct((N_OBJ,), jnp.float32),
                  jax.ShapeDtypeStruct((N_OBJ, DH), jnp.float32),
                  jax.ShapeDtypeStruct((N_OBJ, DH), jnp.float32),
                  jax.ShapeDtypeStruct((N_OBJ * N_OBJ,), jnp.int32),
                  jax.ShapeDtypeStruct((N_REL,), jnp.float32)],
        scratch_types=[
            pltpu.VMEM((CHUNK,), jnp.int32),
            pltpu.VMEM((CHUNK,), jnp.int32),
            pltpu.VMEM((CHUNK,), jnp.int32),
            pltpu.VMEM((CHUNK + L,), jnp.float32),
            pltpu.VMEM((D + L,), jnp.float32),
            pltpu.VMEM((L,), jnp.int32),
            pltpu.VMEM((NT, L), jnp.int32),
            pltpu.VMEM((L,), jnp.int32),
            pltpu.VMEM((L,), jnp.int32),
            pltpu.VMEM((L,), jnp.int32),
            pltpu.VMEM((L, D), jnp.float32),
            pltpu.VMEM((L, D), jnp.float32),
            pltpu.VMEM((L, D), jnp.float32),
            pltpu.VMEM((L,), jnp.int32),
            pltpu.VMEM((L,), jnp.int32),
            pltpu.VMEM((L,), jnp.float32),
            pltpu.VMEM((2 * L,), jnp.float32),
            pltpu.VMEM((L, DH), jnp.float32),
            pltpu.VMEM((L, DH), jnp.float32),
            pltpu.VMEM((L, DH), jnp.float32),
            pltpu.VMEM((L, DH), jnp.float32),
            pltpu.VMEM((L,), jnp.float32),
            pltpu.VMEM((L, DH), jnp.float32),
            pltpu.VMEM((ROWS_T,), jnp.float32),
            pltpu.VMEM_SHARED((N_OBJ,), jnp.float32),
            pltpu.VMEM_SHARED((N_OBJ, DH), jnp.float32),
            pltpu.VMEM_SHARED((N_OBJ, DH), jnp.float32),
            pltpu.VMEM_SHARED((NT, L), jnp.int32),
            pltpu.VMEM_SHARED((ZCAP,), jnp.float32),
            pltpu.SemaphoreType.DMA,
            pltpu.SemaphoreType.DMA,
            pltpu.SemaphoreType.DMA,
        ])
    return f(rel0, rel1, rel2, s, o, phr, convf, wwb)


def kernel(obj_feats, phr_feats, im_inds, rel_inds, ws_w, ws_b, wo_w, wo_b,
           w_w, w_b, conv_w, conv_b, t1_w, t1_b, ln_g, ln_b, t2_w, t2_b):
    s, o, convf = pl.pallas_call(
        _tc1_body,
        out_shape=[jax.ShapeDtypeStruct((N_OBJ, D), jnp.float32),
                   jax.ShapeDtypeStruct((N_OBJ, D), jnp.float32),
                   jax.ShapeDtypeStruct((N_OBJ, DH), jnp.float32)],
    )(obj_feats, ws_w, ws_b, wo_w, wo_b, conv_w, conv_b)

    rel0 = rel_inds[:, 0]
    rel1 = rel_inds[:, 1]
    rel2 = rel_inds[:, 2]
    wwb = jnp.concatenate([w_w[:, 0], w_b,
                           jnp.zeros((L - 1,), jnp.float32)])

    rsum, out1c, out2c, _own, _zs = _sc_call(rel0, rel1, rel2, s, o,
                                             phr_feats, convf, wwb)

    out = pl.pallas_call(
        _tc2_body,
        out_shape=jax.ShapeDtypeStruct((N_OBJ, D), jnp.float32),
    )(obj_feats, convf, rsum, out1c, out2c, t1_w, t1_b, ln_g, ln_b,
      t2_w, t2_b)
    return out
